# Initial kernel scaffold; baseline (speedup 1.0000x reference)
import jax
import jax.numpy as jnp
from jax.experimental import pallas as pl


def kernel(x, edge_index, W1, b1, W2, b2):
    raise NotImplementedError("write your pallas kernel here")


# speedup vs baseline: 12.7592x; 12.7592x over previous
import functools

import jax
import jax.numpy as jnp
from jax import lax
from jax.experimental import pallas as pl
from jax.experimental.pallas import tpu as pltpu
from jax.experimental.pallas import tpu_sc as plsc

NC = 2
NS = 16
NW = NC * NS
L = 16
CHUNK = 128


def _mesh():
    return plsc.VectorSubcoreMesh(
        core_axis_name="c", subcore_axis_name="s",
        num_cores=NC, num_subcores=NS)


def _zero_rows(buf, acc_sh, r0, rpt):
    n_full = rpt // CHUNK
    rem = rpt % CHUNK
    for k in range(n_full):
        pltpu.sync_copy(buf, acc_sh.at[pl.ds(r0 + k * CHUNK, CHUNK)])
    if rem:
        pltpu.sync_copy(buf.at[pl.ds(0, rem)],
                        acc_sh.at[pl.ds(r0 + n_full * CHUNK, rem)])


def _sc_degree(dst_p, n_pad, nt):
    nchunk = nt // CHUNK

    @functools.partial(
        pl.kernel,
        out_type=jax.ShapeDtypeStruct((NC * n_pad, L), jnp.float32),
        mesh=_mesh(),
        compiler_params=pltpu.CompilerParams(use_tc_tiling_on_sc=False),
        scratch_types=[
            pltpu.VMEM((CHUNK,), jnp.int32),
            pltpu.VMEM((CHUNK, L), jnp.float32),
            pltpu.VMEM_SHARED((n_pad, L), jnp.float32),
        ],
    )
    def deg_kernel(dst_hbm, out_hbm, idx_v, ones_v, acc_sh):
        c = lax.axis_index("c")
        s = lax.axis_index("s")
        wid = c * NS + s
        rpt = n_pad // NS
        r0 = s * rpt

        @pl.loop(0, CHUNK)
        def _(r):
            ones_v[r, pl.ds(0, L)] = jnp.zeros((L,), jnp.float32)

        _zero_rows(ones_v, acc_sh, r0, rpt)

        @pl.loop(0, CHUNK)
        def _(r):
            ones_v[r, pl.ds(0, L)] = jnp.full((L,), 1.0, jnp.float32)

        plsc.subcore_barrier()

        @pl.loop(0, nchunk)
        def _(k):
            base = wid * nt + k * CHUNK
            pltpu.sync_copy(dst_hbm.at[pl.ds(base, CHUNK)], idx_v)
            pltpu.sync_copy(ones_v, acc_sh.at[idx_v], add=True)

        plsc.subcore_barrier()
        pltpu.sync_copy(acc_sh.at[pl.ds(r0, rpt)],
                        out_hbm.at[pl.ds(c * n_pad + r0, rpt)])

    return deg_kernel(dst_p)


def _sc_agg(hs, src_p, dst_p, n_pad, nt):
    d_dim = hs.shape[1]
    nchunk = nt // CHUNK

    @functools.partial(
        pl.kernel,
        out_type=jax.ShapeDtypeStruct((NC * n_pad, d_dim), jnp.float32),
        mesh=_mesh(),
        compiler_params=pltpu.CompilerParams(use_tc_tiling_on_sc=False),
        scratch_types=[
            pltpu.VMEM((CHUNK,), jnp.int32),
            pltpu.VMEM((CHUNK,), jnp.int32),
            pltpu.VMEM((CHUNK, d_dim), jnp.float32),
            pltpu.VMEM_SHARED((n_pad, d_dim), jnp.float32),
            pltpu.SemaphoreType.DMA,
        ],
    )
    def agg_kernel(hs_hbm, src_hbm, dst_hbm, out_hbm,
                   src_v, dst_v, rows_v, acc_sh, sem):
        c = lax.axis_index("c")
        s = lax.axis_index("s")
        wid = c * NS + s
        rpt = n_pad // NS
        r0 = s * rpt

        @pl.loop(0, CHUNK)
        def _(r):
            @pl.loop(0, d_dim, step=L)
            def _(j):
                rows_v[r, pl.ds(j, L)] = jnp.zeros((L,), jnp.float32)

        _zero_rows(rows_v, acc_sh, r0, rpt)
        plsc.subcore_barrier()

        @pl.loop(0, nchunk)
        def _(k):
            base = wid * nt + k * CHUNK
            pltpu.sync_copy(src_hbm.at[pl.ds(base, CHUNK)], src_v)
            pltpu.sync_copy(dst_hbm.at[pl.ds(base, CHUNK)], dst_v)
            pltpu.async_copy(hs_hbm.at[src_v], rows_v, sem).wait()
            pltpu.sync_copy(rows_v, acc_sh.at[dst_v], add=True)

        plsc.subcore_barrier()
        pltpu.sync_copy(acc_sh.at[pl.ds(r0, rpt)],
                        out_hbm.at[pl.ds(c * n_pad + r0, rpt)])

    return agg_kernel(hs, src_p, dst_p)


BN = 1000


def _mm(x, w):
    n, k = x.shape
    m = w.shape[1]

    def body(x_ref, w_ref, o_ref):
        o_ref[...] = jnp.dot(x_ref[...], w_ref[...],
                             preferred_element_type=jnp.float32)

    return pl.pallas_call(
        body,
        grid=(n // BN,),
        in_specs=[pl.BlockSpec((BN, k), lambda i: (i, 0)),
                  pl.BlockSpec((k, m), lambda i: (0, 0))],
        out_specs=pl.BlockSpec((BN, m), lambda i: (i, 0)),
        out_shape=jax.ShapeDtypeStruct((n, m), jnp.float32),
    )(x, w)


def _dis_block(d0_ref, d1_ref):
    return lax.rsqrt(d0_ref[:, 0:1] + d1_ref[:, 0:1] + 1.0)


def _scale(h1, d0, d1):
    n, m = h1.shape

    def body(h_ref, d0_ref, d1_ref, o_ref):
        o_ref[...] = h_ref[...] * _dis_block(d0_ref, d1_ref)

    return pl.pallas_call(
        body,
        grid=(n // BN,),
        in_specs=[pl.BlockSpec((BN, m), lambda i: (i, 0)),
                  pl.BlockSpec((BN, L), lambda i: (i, 0)),
                  pl.BlockSpec((BN, L), lambda i: (i, 0))],
        out_specs=pl.BlockSpec((BN, m), lambda i: (i, 0)),
        out_shape=jax.ShapeDtypeStruct((n, m), jnp.float32),
    )(h1, d0, d1)


def _combine_mm(p0, p1, hs1, d0, d1, b1, w2):
    n, m = hs1.shape
    m2 = w2.shape[1]

    def body(p0_ref, p1_ref, hs_ref, d0_ref, d1_ref, b_ref, w_ref, o_ref):
        dis = _dis_block(d0_ref, d1_ref)
        pre = dis * (p0_ref[...] + p1_ref[...] + hs_ref[...]) + b_ref[...]
        h2 = jnp.maximum(pre, 0.0)
        o_ref[...] = dis * jnp.dot(h2, w_ref[...],
                                   preferred_element_type=jnp.float32)

    return pl.pallas_call(
        body,
        grid=(n // BN,),
        in_specs=[pl.BlockSpec((BN, m), lambda i: (i, 0)),
                  pl.BlockSpec((BN, m), lambda i: (i, 0)),
                  pl.BlockSpec((BN, m), lambda i: (i, 0)),
                  pl.BlockSpec((BN, L), lambda i: (i, 0)),
                  pl.BlockSpec((BN, L), lambda i: (i, 0)),
                  pl.BlockSpec((1, m), lambda i: (0, 0)),
                  pl.BlockSpec((m, m2), lambda i: (0, 0))],
        out_specs=pl.BlockSpec((BN, m2), lambda i: (i, 0)),
        out_shape=jax.ShapeDtypeStruct((n, m2), jnp.float32),
    )(p0, p1, hs1, d0, d1, b1.reshape(1, m), w2)


def _final(p0, p1, hs2, d0, d1, b2):
    n, m = hs2.shape

    def body(p0_ref, p1_ref, hs_ref, d0_ref, d1_ref, b_ref, o_ref):
        dis = _dis_block(d0_ref, d1_ref)
        o_ref[...] = dis * (p0_ref[...] + p1_ref[...] + hs_ref[...]) + b_ref[...]

    return pl.pallas_call(
        body,
        grid=(n // BN,),
        in_specs=[pl.BlockSpec((BN, m), lambda i: (i, 0)),
                  pl.BlockSpec((BN, m), lambda i: (i, 0)),
                  pl.BlockSpec((BN, m), lambda i: (i, 0)),
                  pl.BlockSpec((BN, L), lambda i: (i, 0)),
                  pl.BlockSpec((BN, L), lambda i: (i, 0)),
                  pl.BlockSpec((1, m), lambda i: (0, 0))],
        out_specs=pl.BlockSpec((BN, m), lambda i: (i, 0)),
        out_shape=jax.ShapeDtypeStruct((n, m), jnp.float32),
    )(p0, p1, hs2, d0, d1, b2.reshape(1, m))


def kernel(x, edge_index, W1, b1, W2, b2):
    n = x.shape[0]
    e = edge_index.shape[1]
    src = edge_index[0].astype(jnp.int32)
    dst = edge_index[1].astype(jnp.int32)

    nt = -(-e // NW)
    nt = -(-nt // CHUNK) * CHUNK
    e_pad = nt * NW
    n_pad = (n // (NS * 8) + 1) * NS * 8

    pad = e_pad - e
    src_p = jnp.concatenate([src, jnp.zeros((pad,), jnp.int32)])
    dst_p = jnp.concatenate([dst, jnp.full((pad,), n_pad - 1, jnp.int32)])

    degp = _sc_degree(dst_p, n_pad, nt)
    h1 = _mm(x, W1)
    d0 = degp[:n]
    d1 = degp[n_pad:n_pad + n]

    hs1 = _scale(h1, d0, d1)
    a1 = _sc_agg(hs1, src_p, dst_p, n_pad, nt)
    hs2 = _combine_mm(a1[:n], a1[n_pad:n_pad + n], hs1, d0, d1, b1, W2)
    a2 = _sc_agg(hs2, src_p, dst_p, n_pad, nt)
    return _final(a2[:n], a2[n_pad:n_pad + n], hs2, d0, d1, b2)
